# baseline (speedup 1.0000x reference)
import jax
import jax.numpy as jnp
from jax import lax
from jax.experimental import pallas as pl
from jax.experimental.pallas import tpu as pltpu
from jax.experimental.pallas import tpu_sc as plsc

B = 4096
L = 200
DIM = 64

NC = 2
NS = 16
NW = NC * NS
ROWS_PER_W = B // NW
ROWS_PER_CHUNK = 2
CHUNK = ROWS_PER_CHUNK * L
NBUF = 4
STEPS = ROWS_PER_W // ROWS_PER_CHUNK


def _sc_gather(table, idx):
    mesh = plsc.VectorSubcoreMesh(core_axis_name="c", subcore_axis_name="s")

    @pl.kernel(
        out_type=jax.ShapeDtypeStruct((B, L, DIM), jnp.float32),
        mesh=mesh,
        scratch_types=[
            pltpu.VMEM((NBUF, CHUNK), jnp.int32),
            pltpu.VMEM((NBUF, CHUNK, DIM), jnp.float32),
            pltpu.SemaphoreType.DMA((NBUF,)),
        ],
        compiler_params=pltpu.CompilerParams(use_tc_tiling_on_sc=False),
    )
    def gather_kernel(table_hbm, idx_hbm, out_hbm, idx_v, rows_v, sems):
        wid = lax.axis_index("s") * NC + lax.axis_index("c")
        base = wid * ROWS_PER_W

        def fire(b, chunk_i):
            row = base + chunk_i * ROWS_PER_CHUNK
            for r in range(ROWS_PER_CHUNK):
                pltpu.sync_copy(idx_hbm.at[row + r],
                                idx_v.at[b, pl.ds(r * L, L)])
            pltpu.async_copy(table_hbm.at[idx_v.at[b]], rows_v.at[b],
                             sems.at[b])

        def drain_store(b, chunk_i):
            pltpu.make_async_copy(table_hbm.at[idx_v.at[b]], rows_v.at[b],
                                  sems.at[b]).wait()
            row = base + chunk_i * ROWS_PER_CHUNK
            for r in range(ROWS_PER_CHUNK):
                pltpu.sync_copy(rows_v.at[b, pl.ds(r * L, L)],
                                out_hbm.at[row + r])

        for b in range(NBUF):
            fire(b, b)

        @pl.loop(0, STEPS // NBUF - 1)
        def _(h):
            for b in range(NBUF):
                i = h * NBUF + b
                drain_store(b, i)
                fire(b, i + NBUF)

        for b in range(NBUF):
            drain_store(b, STEPS - NBUF + b)

    return gather_kernel(table, idx)


def kernel(x, table):
    return _sc_gather(table, x.astype(jnp.int32))


# speedup vs baseline: 1.0328x; 1.0328x over previous
import jax
import jax.numpy as jnp
from jax import lax
from jax.experimental import pallas as pl
from jax.experimental.pallas import tpu as pltpu
from jax.experimental.pallas import tpu_sc as plsc

B = 4096
L = 200
DIM = 64
N = B * L
VROWS = 500000

NC = 2
NS = 16
NW = NC * NS

ROWS_PER_W = N // NW
K = 128
NCHUNK = ROWS_PER_W // K


def _sc_gather(tbl2, ridx, hoff):
    mesh = plsc.VectorSubcoreMesh(core_axis_name="c", subcore_axis_name="s")

    @pl.kernel(
        out_type=jax.ShapeDtypeStruct((N, DIM), jnp.float32),
        mesh=mesh,
        scratch_types=[
            pltpu.VMEM((K,), jnp.int32),
            pltpu.VMEM((K,), jnp.int32),
            pltpu.VMEM((2, K), jnp.int32),
            pltpu.VMEM((2, K, 128), jnp.float32),
            pltpu.VMEM((2, K, DIM), jnp.float32),
            pltpu.SemaphoreType.DMA((2,)),
            pltpu.SemaphoreType.DMA((2,)),
            pltpu.SemaphoreType.DMA((2,)),
            pltpu.SemaphoreType.DMA((2,)),
        ],
    )
    def gather_kernel(tbl_hbm, ridx_hbm, hoff_hbm, out_hbm,
                      ridx0_v, ridx1_v, hoff_v, pairs_v, out_v,
                      sem_i, sem_h, sem_g, sem_o):
        wid = lax.axis_index("s") * NC + lax.axis_index("c")
        row0 = wid * ROWS_PER_W
        ridx_bufs = (ridx0_v, ridx1_v)

        def load_ridx(b, c):
            pltpu.async_copy(ridx_hbm.at[pl.ds(row0 + c * K, K)],
                             ridx_bufs[b], sem_i.at[b])

        def wait_ridx(b):
            pltpu.make_async_copy(ridx_hbm.at[pl.ds(0, K)], ridx_bufs[b],
                                  sem_i.at[b]).wait()

        def load_hoff(b, c):
            pltpu.async_copy(hoff_hbm.at[pl.ds(row0 + c * K, K)],
                             hoff_v.at[b], sem_h.at[b])

        def wait_hoff(b):
            pltpu.make_async_copy(hoff_hbm.at[pl.ds(0, K)], hoff_v.at[b],
                                  sem_h.at[b]).wait()

        def fire_gather(b):
            pltpu.async_copy(tbl_hbm.at[ridx_bufs[b]], pairs_v.at[b],
                             sem_g.at[b])

        def wait_gather(b):
            pltpu.make_async_copy(tbl_hbm.at[pl.ds(0, K)], pairs_v.at[b],
                                  sem_g.at[b]).wait()

        def select(b):
            @pl.loop(0, K // 16)
            def _(g):
                hvec = hoff_v[b, pl.ds(g * 16, 16)]
                for j in range(16):
                    r = g * 16 + j
                    h = hvec[j]
                    for m in range(DIM // 16):
                        out_v[b, r, pl.ds(16 * m, 16)] = (
                            pairs_v[b, r, pl.ds(h + 16 * m, 16)])

        def fire_store(b, c):
            pltpu.async_copy(out_v.at[b],
                             out_hbm.at[pl.ds(row0 + c * K, K)], sem_o.at[b])

        def wait_store(b):
            pltpu.make_async_copy(out_v.at[b], out_hbm.at[pl.ds(0, K)],
                                  sem_o.at[b]).wait()

        load_ridx(0, 0)
        load_hoff(0, 0)
        load_ridx(1, 1)
        load_hoff(1, 1)
        wait_ridx(0)
        fire_gather(0)
        wait_ridx(1)
        fire_gather(1)
        wait_gather(0)
        load_ridx(0, 2)
        wait_hoff(0)
        select(0)
        fire_store(0, 0)
        load_hoff(0, 2)
        wait_ridx(0)
        fire_gather(0)
        wait_gather(1)
        load_ridx(1, 3)
        wait_hoff(1)
        select(1)
        fire_store(1, 1)
        load_hoff(1, 3)

        @pl.loop(1, NCHUNK // 2 - 1)
        def _(h):
            c0 = 2 * h
            wait_store(1)
            wait_ridx(1)
            fire_gather(1)
            wait_gather(0)
            load_ridx(0, c0 + 2)
            wait_store(0)
            wait_hoff(0)
            select(0)
            fire_store(0, c0)
            load_hoff(0, c0 + 2)
            wait_ridx(0)
            fire_gather(0)
            wait_gather(1)
            load_ridx(1, c0 + 3)
            wait_hoff(1)
            select(1)
            fire_store(1, c0 + 1)
            load_hoff(1, c0 + 3)

        wait_store(1)
        wait_ridx(1)
        fire_gather(1)
        wait_gather(0)
        wait_store(0)
        wait_hoff(0)
        select(0)
        fire_store(0, NCHUNK - 2)
        wait_gather(1)
        wait_hoff(1)
        select(1)
        fire_store(1, NCHUNK - 1)
        wait_store(0)
        wait_store(1)

    return gather_kernel(tbl2, ridx, hoff)


def kernel(x, table):
    idx = x.astype(jnp.int32).reshape(N)
    ridx = idx >> 1
    hoff = (idx & 1) * DIM
    tbl2 = table.reshape(VROWS, 2 * DIM)
    out = _sc_gather(tbl2, ridx, hoff)
    return out.reshape(B, L, DIM)
